# Initial kernel scaffold; baseline (speedup 1.0000x reference)
import jax
import jax.numpy as jnp
from jax.experimental import pallas as pl


def kernel(x):
    raise NotImplementedError("write your pallas kernel here")


# speedup vs baseline: 1.2109x; 1.2109x over previous
import jax
import jax.numpy as jnp
import numpy as np
from jax import lax
from jax.experimental import pallas as pl
from jax.experimental.pallas import tpu as pltpu
from jax.experimental.pallas import tpu_sc as plsc

_N = 32768
_K = 8192
_L = 16
_NW = 16
_CHUNK = _N // _NW
_NV = _CHUNK // _L
_NB = 256
_NBV = _NB // _L

_MININT = np.int32(-(2**31))
_M31 = np.int32(0x7FFFFFFF)


def _srl(x, amount):
    return lax.shift_right_logical(x, jnp.full(x.shape, amount, jnp.int32))


def _sc_body(x_hbm, out_hbm, xv, ub, hist, gh, merged, shist, finv, sfin,
             fmerged, outv):
    sid = lax.axis_index("s")
    cid = lax.axis_index("c")
    base = sid * _CHUNK
    pltpu.sync_copy(x_hbm.at[pl.ds(base, _CHUNK)], xv)

    ones = jnp.ones((_L,), jnp.int32)

    def conv(i, c):
        xb = xv[pl.ds(i * _L, _L)]
        ib = plsc.bitcast(xb, jnp.int32)
        ks = jnp.where(ib >= 0, ib, ib ^ _M31)
        ub[pl.ds(i * _L, _L)] = ks ^ _MININT
        return c

    lax.fori_loop(0, _NV, conv, jnp.int32(0))

    phi = jnp.int32(0)
    kr = jnp.int32(_K)

    for p in (3, 2, 1, 0):
        shift = 8 * p

        def zb(i, c):
            hist[pl.ds(i * _L, _L)] = jnp.zeros((_L,), jnp.int32)
            return c

        lax.fori_loop(0, _NBV, zb, jnp.int32(0))

        if p == 3:
            def hb(i, c):
                u = ub[pl.ds(i * _L, _L)]
                f = _srl(u, 24) & 0xFF
                plsc.addupdate_scatter(hist, [f], ones)
                return c
        else:
            def hb(i, c, _shift=shift, _phi=phi):
                u = ub[pl.ds(i * _L, _L)]
                act = _srl(u, _shift + 8) == _phi
                f = _srl(u, _shift) & 0xFF
                plsc.addupdate_scatter(hist, [f], ones, mask=act)
                return c

        lax.fori_loop(0, _NV, hb, jnp.int32(0))

        pltpu.sync_copy(hist, shist.at[pl.ds(sid * _NB, _NB)])
        plsc.subcore_barrier()
        pltpu.sync_copy(shist, merged)
        plsc.subcore_barrier()

        def mg(v, c):
            acc = jnp.zeros((_L,), jnp.int32)
            for w in range(_NW):
                acc = acc + merged[pl.ds(v * _L + w * _NB, _L)]
            gh[pl.ds(v * _L, _L)] = acc
            return c

        lax.fori_loop(0, _NBV, mg, jnp.int32(0))

        def sc(i, carry, _kr=kr):
            cnt, sel, tot = carry
            v = (_NBV - 1) - i
            h = gh[pl.ds(v * _L, _L)]
            s = lax.rev(plsc.cumsum(lax.rev(h, (0,))), (0,)) + tot
            cond = s >= _kr
            cnt = cnt + jnp.sum(jnp.where(cond, 1, 0))
            sel = sel + jnp.sum(jnp.where(cond, h, 0))
            tot = tot + jnp.sum(h)
            return (cnt, sel, tot)

        cnt, sel, tot = lax.fori_loop(
            0, _NBV, sc, (jnp.int32(0), jnp.int32(0), jnp.int32(0)))
        bstar = cnt - 1
        kr = kr - (tot - sel)
        phi = lax.shift_left(phi, jnp.int32(8)) | bstar

    phis = phi ^ _MININT

    def sm(i, a):
        u = ub[pl.ds(i * _L, _L)]
        xb = xv[pl.ds(i * _L, _L)]
        gt = (u ^ _MININT) > phis
        return a + jnp.where(gt, xb, jnp.float32(0))

    acc = lax.fori_loop(0, _NV, sm, jnp.zeros((_L,), jnp.float32))
    sloc = jnp.sum(acc)

    finv[...] = jnp.full((_L,), sloc, jnp.float32)
    pltpu.sync_copy(finv, sfin.at[pl.ds(sid * _L, _L)])
    plsc.subcore_barrier()
    pltpu.sync_copy(sfin, fmerged)

    stot = jnp.zeros((_L,), jnp.float32)
    for w in range(_NW):
        stot = stot + fmerged[pl.ds(w * _L, _L)]

    pv = jnp.full((_L,), phi, jnp.int32)
    ksv = pv ^ _MININT
    bits = jnp.where(ksv >= 0, ksv, ksv ^ _M31)
    tv = plsc.bitcast(bits, jnp.float32)
    krf = jnp.full((_L,), kr, jnp.int32).astype(jnp.float32)
    res = (stot + krf * tv) * jnp.float32(1.0 / _K)

    @pl.when((cid == 0) & (sid == 0))
    def _():
        outv[...] = res
        pltpu.sync_copy(outv, out_hbm)


def _make_kernel():
    mesh = plsc.VectorSubcoreMesh(core_axis_name="c", subcore_axis_name="s")
    return pl.kernel(
        _sc_body,
        out_type=jax.ShapeDtypeStruct((_L,), jnp.float32),
        mesh=mesh,
        compiler_params=pltpu.CompilerParams(needs_layout_passes=False),
        scratch_types=[
            pltpu.VMEM((_CHUNK,), jnp.float32),
            pltpu.VMEM((_CHUNK,), jnp.int32),
            pltpu.VMEM((_NB,), jnp.int32),
            pltpu.VMEM((_NB,), jnp.int32),
            pltpu.VMEM((_NW * _NB,), jnp.int32),
            pltpu.VMEM_SHARED((_NW * _NB,), jnp.int32),
            pltpu.VMEM((_L,), jnp.float32),
            pltpu.VMEM_SHARED((_NW * _L,), jnp.float32),
            pltpu.VMEM((_NW * _L,), jnp.float32),
            pltpu.VMEM((_L,), jnp.float32),
        ],
    )


def kernel(x):
    out = _make_kernel()(x)
    return out[0]


# speedup vs baseline: 1.2634x; 1.0434x over previous
import jax
import jax.numpy as jnp
import numpy as np
from jax import lax
from jax.experimental import pallas as pl
from jax.experimental.pallas import tpu as pltpu
from jax.experimental.pallas import tpu_sc as plsc

_N = 32768
_K = 8192
_L = 16
_NW = 16
_CHUNK = _N // _NW
_NV = _CHUNK // _L
_NB = 256
_NBV = _NB // _L

_MININT = np.int32(-(2**31))
_M31 = np.int32(0x7FFFFFFF)


def _srl(x, amount):
    return lax.shift_right_logical(x, jnp.full(x.shape, amount, jnp.int32))


def _sc_body(x_hbm, out_hbm, xv, ub, hist, gh, merged, shist, finv, sfin,
             fmerged, outv):
    sid = lax.axis_index("s")
    cid = lax.axis_index("c")
    base = sid * _CHUNK
    pltpu.sync_copy(x_hbm.at[pl.ds(base, _CHUNK)], xv)

    ones = jnp.ones((_L,), jnp.int32)

    def conv(i, c):
        xb = xv[pl.ds(i * _L, _L)]
        ib = plsc.bitcast(xb, jnp.int32)
        ks = jnp.where(ib >= 0, ib, ib ^ _M31)
        ub[pl.ds(i * _L, _L)] = ks ^ _MININT
        return c

    lax.fori_loop(0, _NV, conv, jnp.int32(0))

    phi = jnp.int32(0)
    kr = jnp.int32(_K)

    for p in (3, 2, 1, 0):
        shift = 8 * p

        def zb(i, c):
            hist[pl.ds(i * _L, _L)] = jnp.zeros((_L,), jnp.int32)
            return c

        lax.fori_loop(0, _NBV, zb, jnp.int32(0))

        if p == 3:
            def hb(i, c):
                u = ub[pl.ds(i * _L, _L)]
                f = _srl(u, 24) & 0xFF
                plsc.addupdate_scatter(hist, [f], ones)
                return c
        else:
            def hb(i, c, _shift=shift, _phi=phi):
                u = ub[pl.ds(i * _L, _L)]
                act = _srl(u, _shift + 8) == _phi
                f = _srl(u, _shift) & 0xFF
                plsc.addupdate_scatter(hist, [f], ones, mask=act)
                return c

        lax.fori_loop(0, _NV, hb, jnp.int32(0))

        pltpu.sync_copy(hist, shist.at[pl.ds(sid * _NB, _NB)])
        plsc.subcore_barrier()
        pltpu.sync_copy(shist, merged)
        plsc.subcore_barrier()

        def mg(v, c):
            acc = jnp.zeros((_L,), jnp.int32)
            for w in range(_NW):
                acc = acc + merged[pl.ds(v * _L + w * _NB, _L)]
            gh[pl.ds(v * _L, _L)] = acc
            return c

        lax.fori_loop(0, _NBV, mg, jnp.int32(0))

        def sc(i, carry, _kr=kr):
            cnt, sel, tot = carry
            v = (_NBV - 1) - i
            h = gh[pl.ds(v * _L, _L)]
            s = lax.rev(plsc.cumsum(lax.rev(h, (0,))), (0,)) + tot
            cond = s >= _kr
            cnt = cnt + jnp.sum(jnp.where(cond, 1, 0))
            sel = sel + jnp.sum(jnp.where(cond, h, 0))
            tot = tot + jnp.sum(h)
            return (cnt, sel, tot)

        cnt, sel, tot = lax.fori_loop(
            0, _NBV, sc, (jnp.int32(0), jnp.int32(0), jnp.int32(0)))
        bstar = cnt - 1
        kr = kr - (tot - sel)
        phi = lax.shift_left(phi, jnp.int32(8)) | bstar

    phis = phi ^ _MININT

    def sm(i, a):
        u = ub[pl.ds(i * _L, _L)]
        xb = xv[pl.ds(i * _L, _L)]
        gt = (u ^ _MININT) > phis
        return a + jnp.where(gt, xb, jnp.float32(0))

    acc = lax.fori_loop(0, _NV, sm, jnp.zeros((_L,), jnp.float32))
    sloc = jnp.sum(acc)

    finv[...] = jnp.full((_L,), sloc, jnp.float32)
    pltpu.sync_copy(finv, sfin.at[pl.ds(sid * _L, _L)])
    plsc.subcore_barrier()
    pltpu.sync_copy(sfin, fmerged)

    stot = jnp.zeros((_L,), jnp.float32)
    for w in range(_NW):
        stot = stot + fmerged[pl.ds(w * _L, _L)]

    pv = jnp.full((_L,), phi, jnp.int32)
    ksv = pv ^ _MININT
    bits = jnp.where(ksv >= 0, ksv, ksv ^ _M31)
    tv = plsc.bitcast(bits, jnp.float32)
    krf = jnp.full((_L,), kr, jnp.int32).astype(jnp.float32)
    res = (stot + krf * tv) * jnp.float32(1.0 / _K)

    @pl.when((cid == 0) & (sid == 0))
    def _():
        outv[...] = res
        pltpu.sync_copy(outv, out_hbm)


def _make_kernel():
    mesh = plsc.VectorSubcoreMesh(core_axis_name="c", subcore_axis_name="s",
                                  num_cores=1)
    return pl.kernel(
        _sc_body,
        out_type=jax.ShapeDtypeStruct((_L,), jnp.float32),
        mesh=mesh,
        compiler_params=pltpu.CompilerParams(needs_layout_passes=False),
        scratch_types=[
            pltpu.VMEM((_CHUNK,), jnp.float32),
            pltpu.VMEM((_CHUNK,), jnp.int32),
            pltpu.VMEM((_NB,), jnp.int32),
            pltpu.VMEM((_NB,), jnp.int32),
            pltpu.VMEM((_NW * _NB,), jnp.int32),
            pltpu.VMEM_SHARED((_NW * _NB,), jnp.int32),
            pltpu.VMEM((_L,), jnp.float32),
            pltpu.VMEM_SHARED((_NW * _L,), jnp.float32),
            pltpu.VMEM((_NW * _L,), jnp.float32),
            pltpu.VMEM((_L,), jnp.float32),
        ],
    )


def kernel(x):
    out = _make_kernel()(x)
    return out[0]
